# Initial kernel scaffold; baseline (speedup 1.0000x reference)
import functools

import jax
import jax.numpy as jnp
from jax import lax
from jax.experimental import pallas as pl
from jax.experimental.pallas import tpu as pltpu
from jax.experimental.pallas import tpu_sc as plsc

BATCH = 16384
PEPTIDE = 31
VOCAB = 27
EMB = 64

NUM_CORES = 2
NUM_SUBCORES = 16
NUM_WORKERS = NUM_CORES * NUM_SUBCORES
LANES = 16

TOKENS = BATCH * PEPTIDE
TOK_PER_W = TOKENS // NUM_WORKERS
CHUNK = 128
NCHUNKS = TOK_PER_W // CHUNK
VECS_PER_CHUNK = CHUNK // LANES


def _build_combined(aa_table, pos_table):
  def body(aa_ref, pos_ref, out_ref):
    out_ref[...] = aa_ref[:, None, :] + pos_ref[None, :, :]

  return pl.pallas_call(
      body,
      out_shape=jax.ShapeDtypeStruct((VOCAB, PEPTIDE, EMB), jnp.float32),
  )(aa_table, pos_table)


def _make_sc_gather():
  mesh = plsc.VectorSubcoreMesh(core_axis_name="c", subcore_axis_name="s")

  @functools.partial(
      pl.kernel,
      mesh=mesh,
      out_type=jax.ShapeDtypeStruct((TOKENS, EMB), jnp.float32),
      scratch_types=[
          pltpu.VMEM((NCHUNKS, CHUNK), jnp.int32),
          pltpu.VMEM((CHUNK, EMB), jnp.float32),
          pltpu.VMEM((CHUNK, EMB), jnp.float32),
          pltpu.SemaphoreType.DMA,
          pltpu.SemaphoreType.DMA,
      ],
  )
  def sc_gather(c_hbm, x_hbm, out_hbm, idxv, buf0, buf1, sem0, sem1):
    wid = lax.axis_index("s") * NUM_CORES + lax.axis_index("c")
    row0 = wid * NCHUNKS
    tok0 = wid * TOK_PER_W

    pltpu.sync_copy(x_hbm.at[pl.ds(row0, NCHUNKS)], idxv)

    iota = lax.iota(jnp.int32, LANES)

    def compute_idx(r, carry):
      for k in range(VECS_PER_CHUNK):
        col = k * LANES
        pos = lax.rem(r * CHUNK + col + iota, PEPTIDE)
        idxv[r, pl.ds(col, LANES)] = idxv[r, pl.ds(col, LANES)] * PEPTIDE + pos
      return carry

    lax.fori_loop(0, NCHUNKS, compute_idx, 0)

    bufs = (buf0, buf1)
    sems = (sem0, sem1)

    pltpu.async_copy(c_hbm.at[idxv.at[0]], buf0, sem0)
    pltpu.async_copy(c_hbm.at[idxv.at[1]], buf1, sem1)

    def chunk_body(i, carry):
      for b in range(2):
        c = i * 2 + b
        pltpu.make_async_copy(c_hbm.at[idxv.at[c]], bufs[b], sems[b]).wait()
        pltpu.sync_copy(bufs[b], out_hbm.at[pl.ds(tok0 + c * CHUNK, CHUNK)])

        @pl.when(c + 2 < NCHUNKS)
        def _():
          pltpu.async_copy(c_hbm.at[idxv.at[c + 2]], bufs[b], sems[b])

      return carry

    lax.fori_loop(0, NCHUNKS // 2, chunk_body, 0)

  return sc_gather


def kernel(x, aa_table, pos_table):
  combined = _build_combined(aa_table, pos_table)
  c2 = combined.reshape(VOCAB * PEPTIDE, EMB)
  x2 = x.astype(jnp.int32).reshape(TOKENS // CHUNK, CHUNK)
  out = _make_sc_gather()(c2, x2)
  return out.reshape(BATCH, PEPTIDE, EMB)


# speedup vs baseline: 7.6579x; 7.6579x over previous
import functools

import jax
import jax.numpy as jnp
from jax import lax
from jax.experimental import pallas as pl
from jax.experimental.pallas import tpu as pltpu
from jax.experimental.pallas import tpu_sc as plsc

BATCH = 16384
PEPTIDE = 31
VOCAB = 27
EMB = 64

NUM_CORES = 2
NUM_SUBCORES = 16
NUM_WORKERS = NUM_CORES * NUM_SUBCORES
LANES = 16

TOKENS = BATCH * PEPTIDE
TOK_PER_W = TOKENS // NUM_WORKERS
CHUNK = 128
NCHUNKS = TOK_PER_W // CHUNK
NBUF = 4


def _build_combined(aa_table, pos_table):
  def body(aa_ref, pos_ref, out_ref):
    out_ref[...] = aa_ref[...][:, None, :] + pos_ref[...][None, :, :]

  return pl.pallas_call(
      body,
      out_shape=jax.ShapeDtypeStruct((VOCAB, PEPTIDE, EMB), jnp.float32),
  )(aa_table, pos_table)


def _make_sc_gather():
  mesh = plsc.VectorSubcoreMesh(core_axis_name="c", subcore_axis_name="s")

  return functools.partial(
      pl.kernel,
      mesh=mesh,
      out_type=jax.ShapeDtypeStruct((TOKENS, EMB), jnp.float32),
      compiler_params=pltpu.CompilerParams(use_tc_tiling_on_sc=False),
      scratch_types=[
          pltpu.VMEM((NCHUNKS, CHUNK), jnp.int32),
      ]
      + [pltpu.VMEM((CHUNK, EMB), jnp.float32) for _ in range(NBUF)]
      + [pltpu.SemaphoreType.DMA for _ in range(2 * NBUF)],
  )


def _sc_body(c_hbm, x_hbm, out_hbm, idxv,
             buf0, buf1, buf2, buf3,
             gsem0, gsem1, gsem2, gsem3,
             wsem0, wsem1, wsem2, wsem3):
  bufs = (buf0, buf1, buf2, buf3)
  gsems = (gsem0, gsem1, gsem2, gsem3)
  wsems = (wsem0, wsem1, wsem2, wsem3)

  wid = lax.axis_index("s") * NUM_CORES + lax.axis_index("c")
  tok0 = wid * TOK_PER_W

  pltpu.sync_copy(x_hbm.at[wid], idxv)

  iota = lax.iota(jnp.int32, LANES)

  def build_idx(c, carry):
    for k in range(CHUNK // LANES):
      col = k * LANES
      pos = lax.rem(c * CHUNK + col + iota, PEPTIDE)
      idxv[c, pl.ds(col, LANES)] = idxv[c, pl.ds(col, LANES)] * PEPTIDE + pos
    return carry

  lax.fori_loop(0, NCHUNKS, build_idx, 0)

  def start_gather(c, b):
    pltpu.async_copy(c_hbm.at[idxv.at[c]], bufs[b], gsems[b])

  def wait_gather(c, b):
    pltpu.make_async_copy(c_hbm.at[idxv.at[c]], bufs[b], gsems[b]).wait()

  def start_write(c, b):
    pltpu.async_copy(bufs[b], out_hbm.at[pl.ds(tok0 + c * CHUNK, CHUNK)],
                     wsems[b])

  def drain_write(b):
    pltpu.make_async_copy(bufs[b], out_hbm.at[pl.ds(tok0, CHUNK)],
                          wsems[b]).wait()

  start_gather(0, 0)
  start_gather(1, 1)

  def chunk_step(i, carry):
    for b in range(NBUF):
      c = i * NBUF + b
      bn = (b + 2) % NBUF

      @pl.when(jnp.logical_and(c >= 2, c + 2 < NCHUNKS))
      def _():
        drain_write(bn)

      @pl.when(c + 2 < NCHUNKS)
      def _():
        start_gather(c + 2, bn)

      wait_gather(c, b)
      start_write(c, b)
    return carry

  lax.fori_loop(0, NCHUNKS // NBUF, chunk_step, 0)

  for b in range(NBUF):
    drain_write(b)


def kernel(x, aa_table, pos_table):
  combined = _build_combined(aa_table, pos_table)
  c2 = combined.reshape(VOCAB * PEPTIDE, EMB)
  x2 = x.astype(jnp.int32).reshape(NUM_WORKERS, NCHUNKS, CHUNK)
  out = _make_sc_gather()(_sc_body)(c2, x2)
  return out.reshape(BATCH, PEPTIDE, EMB)


# speedup vs baseline: 9.0810x; 1.1858x over previous
import functools

import jax
import jax.numpy as jnp
from jax import lax
from jax.experimental import pallas as pl
from jax.experimental.pallas import tpu as pltpu
from jax.experimental.pallas import tpu_sc as plsc

BATCH = 16384
PEPTIDE = 31
VOCAB = 27
EMB = 64

NUM_CORES = 2
NUM_SUBCORES = 16
NUM_WORKERS = NUM_CORES * NUM_SUBCORES
LANES = 16

TOKENS = BATCH * PEPTIDE
TOK_PER_W = TOKENS // NUM_WORKERS
CHUNK = 128
NCHUNKS = TOK_PER_W // CHUNK
NBUF = 4


def _build_combined(aa_table, pos_table):
  def body(aa_ref, pos_ref, out_ref):
    out_ref[...] = aa_ref[...][:, None, :] + pos_ref[...][None, :, :]

  return pl.pallas_call(
      body,
      out_shape=jax.ShapeDtypeStruct((VOCAB, PEPTIDE, EMB), jnp.float32),
  )(aa_table, pos_table)


def _make_sc_gather():
  mesh = plsc.VectorSubcoreMesh(core_axis_name="c", subcore_axis_name="s")

  return functools.partial(
      pl.kernel,
      mesh=mesh,
      out_type=jax.ShapeDtypeStruct((TOKENS, EMB), jnp.float32),
      compiler_params=pltpu.CompilerParams(use_tc_tiling_on_sc=False),
      scratch_types=[
          pltpu.VMEM((NCHUNKS, CHUNK), jnp.int32),
      ]
      + [pltpu.VMEM((CHUNK, EMB), jnp.float32) for _ in range(NBUF)]
      + [pltpu.SemaphoreType.DMA for _ in range(2 * NBUF)],
  )


def _sc_body(c_hbm, x_hbm, out_hbm, idxv,
             buf0, buf1, buf2, buf3,
             gsem0, gsem1, gsem2, gsem3,
             wsem0, wsem1, wsem2, wsem3):
  bufs = (buf0, buf1, buf2, buf3)
  gsems = (gsem0, gsem1, gsem2, gsem3)
  wsems = (wsem0, wsem1, wsem2, wsem3)

  wid = lax.axis_index("s") * NUM_CORES + lax.axis_index("c")
  tok0 = wid * TOK_PER_W
  tbase = wid * (VOCAB * PEPTIDE)

  pltpu.sync_copy(x_hbm.at[wid], idxv)

  iota = lax.iota(jnp.int32, LANES)

  def build_idx(c, carry):
    for k in range(CHUNK // LANES):
      col = k * LANES
      pos = lax.rem(c * CHUNK + col + iota, PEPTIDE) + tbase
      idxv[c, pl.ds(col, LANES)] = idxv[c, pl.ds(col, LANES)] * PEPTIDE + pos
    return carry

  lax.fori_loop(0, NCHUNKS, build_idx, 0)

  def start_gather(c, b):
    pltpu.async_copy(c_hbm.at[idxv.at[c]], bufs[b], gsems[b])

  def wait_gather(c, b):
    pltpu.make_async_copy(c_hbm.at[idxv.at[c]], bufs[b], gsems[b]).wait()

  def start_write(c, b):
    pltpu.async_copy(bufs[b], out_hbm.at[pl.ds(tok0 + c * CHUNK, CHUNK)],
                     wsems[b])

  def drain_write(b):
    pltpu.make_async_copy(bufs[b], out_hbm.at[pl.ds(tok0, CHUNK)],
                          wsems[b]).wait()

  start_gather(0, 0)
  start_gather(1, 1)

  def chunk_step(i, carry):
    for b in range(NBUF):
      c = i * NBUF + b
      bn = (b + 2) % NBUF

      @pl.when(jnp.logical_and(c >= 2, c + 2 < NCHUNKS))
      def _():
        drain_write(bn)

      @pl.when(c + 2 < NCHUNKS)
      def _():
        start_gather(c + 2, bn)

      wait_gather(c, b)
      start_write(c, b)
    return carry

  lax.fori_loop(0, NCHUNKS // NBUF, chunk_step, 0)

  for b in range(NBUF):
    drain_write(b)


def kernel(x, aa_table, pos_table):
  combined = _build_combined(aa_table, pos_table)
  c2 = combined.reshape(VOCAB * PEPTIDE, EMB)
  crep = jnp.broadcast_to(
      c2[None], (NUM_WORKERS, VOCAB * PEPTIDE, EMB)).reshape(
          NUM_WORKERS * VOCAB * PEPTIDE, EMB)
  x2 = x.astype(jnp.int32).reshape(NUM_WORKERS, NCHUNKS, CHUNK)
  out = _make_sc_gather()(_sc_body)(crep, x2)
  return out.reshape(BATCH, PEPTIDE, EMB)


# speedup vs baseline: 9.1721x; 1.0100x over previous
import functools

import jax
import jax.numpy as jnp
from jax import lax
from jax.experimental import pallas as pl
from jax.experimental.pallas import tpu as pltpu
from jax.experimental.pallas import tpu_sc as plsc

BATCH = 16384
PEPTIDE = 31
VOCAB = 27
EMB = 64

NUM_CORES = 2
NUM_SUBCORES = 16
NUM_WORKERS = NUM_CORES * NUM_SUBCORES
LANES = 16

TROWS = VOCAB * PEPTIDE
TOKENS = BATCH * PEPTIDE
TOK_PER_W = TOKENS // NUM_WORKERS
CHUNK = 128
NCHUNKS = TOK_PER_W // CHUNK
NBUF = 4
EVECS = EMB // LANES


def _make_sc_kernel():
  mesh = plsc.VectorSubcoreMesh(core_axis_name="c", subcore_axis_name="s")

  return functools.partial(
      pl.kernel,
      mesh=mesh,
      out_type=jax.ShapeDtypeStruct((TOKENS, EMB), jnp.float32),
      compiler_params=pltpu.CompilerParams(use_tc_tiling_on_sc=False),
      scratch_types=[
          pltpu.HBM((NUM_WORKERS * TROWS, EMB), jnp.float32),
          pltpu.VMEM((TROWS, EMB), jnp.float32),
          pltpu.VMEM((VOCAB, EMB), jnp.float32),
          pltpu.VMEM((PEPTIDE, EMB), jnp.float32),
          pltpu.VMEM((NCHUNKS, CHUNK), jnp.int32),
      ]
      + [pltpu.VMEM((CHUNK, EMB), jnp.float32) for _ in range(NBUF)]
      + [pltpu.SemaphoreType.DMA for _ in range(2 * NBUF)],
  )


def _sc_body(aa_hbm, pos_hbm, x_hbm, out_hbm,
             ctab_hbm, tabv, aav, posv, idxv,
             buf0, buf1, buf2, buf3,
             gsem0, gsem1, gsem2, gsem3,
             wsem0, wsem1, wsem2, wsem3):
  bufs = (buf0, buf1, buf2, buf3)
  gsems = (gsem0, gsem1, gsem2, gsem3)
  wsems = (wsem0, wsem1, wsem2, wsem3)

  wid = lax.axis_index("s") * NUM_CORES + lax.axis_index("c")
  tok0 = wid * TOK_PER_W
  tbase = wid * TROWS

  pltpu.sync_copy(x_hbm.at[wid], idxv)
  pltpu.sync_copy(aa_hbm, aav)
  pltpu.sync_copy(pos_hbm, posv)

  def build_vocab(v, carry):
    avecs = [aav[v, pl.ds(e * LANES, LANES)] for e in range(EVECS)]

    def build_pos(p, carry2):
      for e in range(EVECS):
        tabv[v * PEPTIDE + p, pl.ds(e * LANES, LANES)] = (
            avecs[e] + posv[p, pl.ds(e * LANES, LANES)])
      return carry2

    lax.fori_loop(0, PEPTIDE, build_pos, 0)
    return carry

  lax.fori_loop(0, VOCAB, build_vocab, 0)

  pltpu.sync_copy(tabv, ctab_hbm.at[pl.ds(tbase, TROWS)])

  iota = lax.iota(jnp.int32, LANES)

  def build_idx(c, carry):
    for k in range(CHUNK // LANES):
      col = k * LANES
      pos = lax.rem(c * CHUNK + col + iota, PEPTIDE) + tbase
      idxv[c, pl.ds(col, LANES)] = idxv[c, pl.ds(col, LANES)] * PEPTIDE + pos
    return carry

  lax.fori_loop(0, NCHUNKS, build_idx, 0)

  def start_gather(c, b):
    pltpu.async_copy(ctab_hbm.at[idxv.at[c]], bufs[b], gsems[b])

  def wait_gather(c, b):
    pltpu.make_async_copy(ctab_hbm.at[idxv.at[c]], bufs[b], gsems[b]).wait()

  def start_write(c, b):
    pltpu.async_copy(bufs[b], out_hbm.at[pl.ds(tok0 + c * CHUNK, CHUNK)],
                     wsems[b])

  def drain_write(b):
    pltpu.make_async_copy(bufs[b], out_hbm.at[pl.ds(tok0, CHUNK)],
                          wsems[b]).wait()

  start_gather(0, 0)
  start_gather(1, 1)

  def chunk_step(i, carry):
    for b in range(NBUF):
      c = i * NBUF + b
      bn = (b + 2) % NBUF

      @pl.when(jnp.logical_and(c >= 2, c + 2 < NCHUNKS))
      def _():
        drain_write(bn)

      @pl.when(c + 2 < NCHUNKS)
      def _():
        start_gather(c + 2, bn)

      wait_gather(c, b)
      start_write(c, b)
    return carry

  lax.fori_loop(0, NCHUNKS // NBUF, chunk_step, 0)

  for b in range(NBUF):
    drain_write(b)


def kernel(x, aa_table, pos_table):
  x2 = x.astype(jnp.int32).reshape(NUM_WORKERS, NCHUNKS, CHUNK)
  out = _make_sc_kernel()(_sc_body)(aa_table, pos_table, x2)
  return out.reshape(BATCH, PEPTIDE, EMB)
